# baseline (speedup 1.0000x reference)
import functools

import jax
import jax.numpy as jnp
from jax import lax
from jax.experimental import pallas as pl
from jax.experimental.pallas import tpu as pltpu
from jax.experimental.pallas import tpu_sc as plsc

_NC = 2
_NS = 16
_NW = _NC * _NS
_L = 16
_MAP = 64


def _build(t, k):
    rows_w = t // _NW
    n_w = rows_w * k
    rows_per_vec = _L // k
    mesh = plsc.VectorSubcoreMesh(
        core_axis_name="c", subcore_axis_name="s",
        num_cores=_NC, num_subcores=_NS)

    @functools.partial(
        pl.kernel,
        out_type=(jax.ShapeDtypeStruct((t, k), jnp.int32),
                  jax.ShapeDtypeStruct((t, k), jnp.int32)),
        mesh=mesh,
        compiler_params=pltpu.CompilerParams(
            needs_layout_passes=False,
            use_tc_tiling_on_sc=False,
            disable_bounds_checks=True,
            disable_semaphore_checks=True,
            skip_device_barrier=True),
        scratch_types=[
            pltpu.VMEM((rows_w, k), jnp.int32),
            pltpu.VMEM((rows_w, k), jnp.int32),
            pltpu.VMEM((rows_w, k), jnp.int32),
            pltpu.VMEM((128,), jnp.int32),
            pltpu.VMEM((128,), jnp.int32),
            pltpu.VMEM((128,), jnp.int32),
        ],
    )
    def dispatch(idx_hbm, devmap_hbm, locmap_hbm, dev_hbm, loc_hbm,
                 idx_v, dev_v, loc_v, devmap_v, locmap_v, packed_v):
        wid = lax.axis_index("s") * _NC + lax.axis_index("c")
        base = wid * rows_w
        pltpu.sync_copy(devmap_hbm, devmap_v.at[pl.ds(0, _MAP)])
        pltpu.sync_copy(locmap_hbm, locmap_v.at[pl.ds(0, _MAP)])
        pltpu.sync_copy(idx_hbm.at[pl.ds(base, rows_w)], idx_v)

        for j in range(_MAP // _L):
            sl = pl.ds(j * _L, _L)
            packed_v[sl] = (devmap_v[sl] << 16) | (locmap_v[sl] & 0xFFFF)

        lane = lax.iota(jnp.int32, _L)
        row_off = lane // k
        col = lane % k

        def step(i, _):
            row = row_off + i * rows_per_vec
            g = plsc.load_gather(
                packed_v, [plsc.load_gather(idx_v, [row, col])])
            plsc.store_scatter(dev_v, [row, col], g >> 16)
            plsc.store_scatter(loc_v, [row, col], (g << 16) >> 16)
            return 0

        lax.fori_loop(0, n_w // _L, step, 0, unroll=8)
        pltpu.sync_copy(dev_v, dev_hbm.at[pl.ds(base, rows_w)])
        pltpu.sync_copy(loc_v, loc_hbm.at[pl.ds(base, rows_w)])

    return dispatch


def kernel(indices_expert, weight1, weight2, device_indices_map,
           local_expert_indices_map):
    t, k = indices_expert.shape
    dev, loc = _build(t, k)(indices_expert.astype(jnp.int32),
                            device_indices_map.astype(jnp.int32),
                            local_expert_indices_map.astype(jnp.int32))
    out_dtype = indices_expert.dtype
    return dev.astype(out_dtype), loc.astype(out_dtype)


# speedup vs baseline: 3.2248x; 3.2248x over previous
import functools

import jax
import jax.numpy as jnp
from jax import lax
from jax.experimental import pallas as pl
from jax.experimental.pallas import tpu as pltpu
from jax.experimental.pallas import tpu_sc as plsc

_NC = 2
_NS = 16
_NW = _NC * _NS
_L = 16
_MAP = 64


def _build(n):
    per_w = n // _NW
    mesh = plsc.VectorSubcoreMesh(
        core_axis_name="c", subcore_axis_name="s",
        num_cores=_NC, num_subcores=_NS)

    @functools.partial(
        pl.kernel,
        out_type=(jax.ShapeDtypeStruct((n,), jnp.int32),
                  jax.ShapeDtypeStruct((n,), jnp.int32)),
        mesh=mesh,
        compiler_params=pltpu.CompilerParams(
            needs_layout_passes=False,
            use_tc_tiling_on_sc=False,
            disable_bounds_checks=True,
            disable_semaphore_checks=True,
            skip_device_barrier=True),
        scratch_types=[
            pltpu.VMEM((per_w,), jnp.int32),
            pltpu.VMEM((per_w,), jnp.int32),
            pltpu.VMEM((per_w,), jnp.int32),
            pltpu.VMEM((128,), jnp.int32),
            pltpu.VMEM((128,), jnp.int32),
            pltpu.VMEM((128,), jnp.int32),
        ],
    )
    def dispatch(idx_hbm, devmap_hbm, locmap_hbm, dev_hbm, loc_hbm,
                 idx_v, dev_v, loc_v, devmap_v, locmap_v, packed_v):
        wid = lax.axis_index("s") * _NC + lax.axis_index("c")
        base = wid * per_w
        pltpu.sync_copy(devmap_hbm, devmap_v.at[pl.ds(0, _MAP)])
        pltpu.sync_copy(locmap_hbm, locmap_v.at[pl.ds(0, _MAP)])
        pltpu.sync_copy(idx_hbm.at[pl.ds(base, per_w)], idx_v)

        for j in range(_MAP // _L):
            sl = pl.ds(j * _L, _L)
            packed_v[sl] = (devmap_v[sl] << 16) | (locmap_v[sl] & 0xFFFF)

        def step(i, _):
            sl = pl.ds(i * _L, _L)
            g = plsc.load_gather(packed_v, [idx_v[sl]])
            dev_v[sl] = g >> 16
            loc_v[sl] = (g << 16) >> 16
            return 0

        lax.fori_loop(0, per_w // _L, step, 0, unroll=8)
        pltpu.sync_copy(dev_v, dev_hbm.at[pl.ds(base, per_w)])
        pltpu.sync_copy(loc_v, loc_hbm.at[pl.ds(base, per_w)])

    return dispatch


def kernel(indices_expert, weight1, weight2, device_indices_map,
           local_expert_indices_map):
    t, k = indices_expert.shape
    n = t * k
    x = indices_expert.astype(jnp.int32)
    tt = t // 128
    flat = x.reshape(tt, 128, k).transpose(0, 2, 1).reshape(n)
    dev, loc = _build(n)(flat,
                         device_indices_map.astype(jnp.int32),
                         local_expert_indices_map.astype(jnp.int32))
    def unflat(a):
        return a.reshape(tt, k, 128).transpose(0, 2, 1).reshape(t, k)
    out_dtype = indices_expert.dtype
    return unflat(dev).astype(out_dtype), unflat(loc).astype(out_dtype)


# speedup vs baseline: 3.7846x; 1.1736x over previous
import functools

import jax
import jax.numpy as jnp
from jax import lax
from jax.experimental import pallas as pl
from jax.experimental.pallas import tpu as pltpu
from jax.experimental.pallas import tpu_sc as plsc

_NC = 2
_NS = 16
_NW = _NC * _NS
_L = 16
_MAP = 64


def _build(n):
    per_w = n // _NW
    mesh = plsc.VectorSubcoreMesh(
        core_axis_name="c", subcore_axis_name="s",
        num_cores=_NC, num_subcores=_NS)

    @functools.partial(
        pl.kernel,
        out_type=(jax.ShapeDtypeStruct((n,), jnp.int32),
                  jax.ShapeDtypeStruct((n,), jnp.int32)),
        mesh=mesh,
        compiler_params=pltpu.CompilerParams(
            needs_layout_passes=False,
            use_tc_tiling_on_sc=False,
            disable_bounds_checks=True,
            disable_semaphore_checks=True,
            skip_device_barrier=True),
        scratch_types=[
            pltpu.VMEM((per_w,), jnp.int32),
            pltpu.VMEM((per_w,), jnp.int32),
            pltpu.VMEM((per_w,), jnp.int32),
            pltpu.VMEM((128,), jnp.int32),
            pltpu.VMEM((128,), jnp.int32),
            pltpu.VMEM((128,), jnp.int32),
        ],
    )
    def dispatch(idx_hbm, devmap_hbm, locmap_hbm, dev_hbm, loc_hbm,
                 idx_v, dev_v, loc_v, devmap_v, locmap_v, packed_v):
        wid = lax.axis_index("s") * _NC + lax.axis_index("c")
        base = wid * per_w
        pltpu.sync_copy(devmap_hbm, devmap_v.at[pl.ds(0, _MAP)])
        pltpu.sync_copy(locmap_hbm, locmap_v.at[pl.ds(0, _MAP)])
        pltpu.sync_copy(idx_hbm.at[pl.ds(base, per_w)], idx_v)

        for j in range(_MAP // _L):
            sl = pl.ds(j * _L, _L)
            packed_v[sl] = (devmap_v[sl] << 16) | (locmap_v[sl] & 0xFFFF)

        @plsc.parallel_loop(0, per_w, _L, unroll=8)
        def _(off):
            sl = pl.ds(off, _L)
            g = plsc.load_gather(packed_v, [idx_v[sl]])
            dev_v[sl] = g >> 16
            loc_v[sl] = (g << 16) >> 16
        pltpu.sync_copy(dev_v, dev_hbm.at[pl.ds(base, per_w)])
        pltpu.sync_copy(loc_v, loc_hbm.at[pl.ds(base, per_w)])

    return dispatch


def kernel(indices_expert, weight1, weight2, device_indices_map,
           local_expert_indices_map):
    t, k = indices_expert.shape
    n = t * k
    x = indices_expert.astype(jnp.int32)
    tt = t // 128
    flat = x.reshape(tt, 128, k).transpose(0, 2, 1).reshape(n)
    dev, loc = _build(n)(flat,
                         device_indices_map.astype(jnp.int32),
                         local_expert_indices_map.astype(jnp.int32))
    def unflat(a):
        return a.reshape(tt, k, 128).transpose(0, 2, 1).reshape(t, k)
    out_dtype = indices_expert.dtype
    return unflat(dev).astype(out_dtype), unflat(loc).astype(out_dtype)
